# baseline (speedup 1.0000x reference)
import functools

import jax
import jax.numpy as jnp
from jax import lax
from jax.experimental import pallas as pl
from jax.experimental.pallas import tpu as pltpu
from jax.experimental.pallas import tpu_sc as plsc

B, N, D = 8, 4096, 256
R = 2 * B
LOG_N = 12


def _roll_l(x, s):
    return pltpu.roll(x, N - s, 1)


def _roll_r(x, s):
    return pltpu.roll(x, s, 1)


def _sort_body(phi_ref, sphi_ref, idx_ref):
    v = phi_ref[...]
    col = lax.broadcasted_iota(jnp.int32, (R, N), 1)
    ix = col
    for k in range(LOG_N):
        desc = ((col >> (k + 1)) & 1) == 1
        for j in range(k, -1, -1):
            s = 1 << j
            upper = (col & s) != 0
            pv = jnp.where(upper, _roll_r(v, s), _roll_l(v, s))
            pi = jnp.where(upper, _roll_r(ix, s), _roll_l(ix, s))
            gt = (v > pv) | ((v == pv) & (ix > pi))
            take_self = ~(gt ^ upper ^ desc)
            v = jnp.where(take_self, v, pv)
            ix = jnp.where(take_self, ix, pi)
    sphi_ref[...] = v
    row = lax.broadcasted_iota(jnp.int32, (R, N), 0)
    idx_ref[...] = ix + (row % B) * N


def _argsort_stacked(phi2):
    return pl.pallas_call(
        _sort_body,
        out_shape=(
            jax.ShapeDtypeStruct((R, N), jnp.float32),
            jax.ShapeDtypeStruct((R, N), jnp.int32),
        ),
    )(phi2)


_NC, _NS = 2, 16
_NW = _NC * _NS
_ROWS_PER_W = (2 * B * N) // _NW // 2
_CH = 128
_NCHUNK = _ROWS_PER_W // _CH


_NBUF = 3


def _gather_kernel(hit_hbm, key_hbm, idx_hbm, hit_out, key_out,
                   idx_v, bufs, gsems, wsems):
    wid = lax.axis_index("s") * _NC + lax.axis_index("c")
    pltpu.sync_copy(idx_hbm.at[pl.ds(wid * 8, 8)], idx_v.at[pl.ds(0, 8)])
    pltpu.sync_copy(idx_hbm.at[pl.ds((B * N) // _CH + wid * 8, 8)],
                    idx_v.at[pl.ds(8, 8)])

    def gather(c):
        src = hit_hbm if c < _NCHUNK else key_hbm
        b = c % _NBUF
        return pltpu.async_copy(src.at[idx_v.at[c]], bufs[b], gsems[b])

    def write(c):
        b = c % _NBUF
        if c < _NCHUNK:
            dst = hit_out.at[pl.ds(wid * _ROWS_PER_W + c * _CH, _CH)]
        else:
            dst = key_out.at[
                pl.ds(wid * _ROWS_PER_W + (c - _NCHUNK) * _CH, _CH)]
        return pltpu.async_copy(bufs[b], dst, wsems[b])

    nch = 2 * _NCHUNK
    gh = [None] * _NBUF
    wh = [None] * _NBUF
    for c in range(nch):
        b = c % _NBUF
        if wh[b] is not None:
            wh[b].wait()
            wh[b] = None
        gh[b] = gather(c)
        if c >= 1:
            bp = (c - 1) % _NBUF
            gh[bp].wait()
            wh[bp] = write(c - 1)
    bl = (nch - 1) % _NBUF
    gh[bl].wait()
    wh[bl] = write(nch - 1)
    for b in range(_NBUF):
        if wh[b] is not None:
            wh[b].wait()


@functools.cache
def _make_gather_rows():
    @functools.partial(
        pl.kernel,
        mesh=plsc.VectorSubcoreMesh(core_axis_name="c", subcore_axis_name="s"),
        out_type=(
            jax.ShapeDtypeStruct((B * N, D), jnp.float32),
            jax.ShapeDtypeStruct((B * N, D), jnp.float32),
        ),
        scratch_types=[
            pltpu.VMEM((2 * _NCHUNK, _CH), jnp.int32),
        ] + [pltpu.VMEM((_CH, D), jnp.float32)] * _NBUF
          + [pltpu.SemaphoreType.DMA] * (2 * _NBUF),
    )
    def _gather_rows(hit_hbm, key_hbm, idx_hbm, hit_out, key_out,
                     idx_v, b0, b1, b2, g0, g1, g2, w0, w1, w2):
        _gather_kernel(hit_hbm, key_hbm, idx_hbm, hit_out, key_out,
                       idx_v, (b0, b1, b2), (g0, g1, g2), (w0, w1, w2))

    return _gather_rows


def kernel(hit_embed, hit_phi, key_embed, key_phi):
    phi2 = jnp.concatenate([hit_phi, key_phi], axis=0)
    sphi, gidx = _argsort_stacked(phi2)
    hit_s, key_s = _make_gather_rows()(
        hit_embed.reshape(B * N, D),
        key_embed.reshape(B * N, D),
        gidx.reshape((2 * B * N) // _CH, _CH),
    )
    return (
        hit_s.reshape(B, N, D),
        sphi[:B],
        key_s.reshape(B, N, D),
        sphi[B:],
    )


# speedup vs baseline: 1.1074x; 1.1074x over previous
import functools

import jax
import jax.numpy as jnp
from jax import lax
from jax.experimental import pallas as pl
from jax.experimental.pallas import tpu as pltpu
from jax.experimental.pallas import tpu_sc as plsc

B, N, D = 8, 4096, 256
LOG_N = 12


def _roll_l(x, s):
    return jnp.concatenate([x[:, s:], x[:, :s]], axis=1)


def _roll_r(x, s):
    return jnp.concatenate([x[:, -s:], x[:, :-s]], axis=1)


def _sort_body(phi_ref, sphi_ref, idx_ref):
    v = phi_ref[...]
    col = lax.broadcasted_iota(jnp.int32, (B, N), 1)
    ix = col
    for k in range(LOG_N):
        desc = ((col >> (k + 1)) & 1) == 1
        for j in range(k, -1, -1):
            s = 1 << j
            upper = (col & s) != 0
            pv = jnp.where(upper, _roll_r(v, s), _roll_l(v, s))
            pi = jnp.where(upper, _roll_r(ix, s), _roll_l(ix, s))
            gt = (v > pv) | ((v == pv) & (ix > pi))
            take_self = ~(gt ^ upper ^ desc)
            v = jnp.where(take_self, v, pv)
            ix = jnp.where(take_self, ix, pi)
    sphi_ref[...] = v
    row = lax.broadcasted_iota(jnp.int32, (B, N), 0)
    idx_ref[...] = ix + row * N


def _argsort_batch(phi):
    return pl.pallas_call(
        _sort_body,
        out_shape=(
            jax.ShapeDtypeStruct((B, N), jnp.float32),
            jax.ShapeDtypeStruct((B, N), jnp.int32),
        ),
    )(phi)


_NC, _NS = 2, 16
_NW = _NC * _NS
_ROWS_PER_W = (B * N) // _NW
_CH = 128
_NCHUNK = _ROWS_PER_W // _CH
_NBUF = 3


def _gather_kernel(tab_hbm, idx_hbm, out_hbm, idx_v, bufs, gsems, wsems):
    wid = lax.axis_index("s") * _NC + lax.axis_index("c")
    pltpu.sync_copy(idx_hbm.at[pl.ds(wid * _NCHUNK, _NCHUNK)], idx_v)

    def gather(c):
        b = c % _NBUF
        return pltpu.async_copy(tab_hbm.at[idx_v.at[c]], bufs[b], gsems[b])

    def write(c):
        b = c % _NBUF
        dst = out_hbm.at[pl.ds(wid * _ROWS_PER_W + c * _CH, _CH)]
        return pltpu.async_copy(bufs[b], dst, wsems[b])

    gh = [None] * _NBUF
    wh = [None] * _NBUF
    for c in range(_NCHUNK):
        b = c % _NBUF
        if wh[b] is not None:
            wh[b].wait()
            wh[b] = None
        gh[b] = gather(c)
        if c >= 1:
            bp = (c - 1) % _NBUF
            gh[bp].wait()
            wh[bp] = write(c - 1)
    bl = (_NCHUNK - 1) % _NBUF
    gh[bl].wait()
    wh[bl] = write(_NCHUNK - 1)
    for b in range(_NBUF):
        if wh[b] is not None:
            wh[b].wait()


@functools.cache
def _make_gather_rows():
    @functools.partial(
        pl.kernel,
        mesh=plsc.VectorSubcoreMesh(core_axis_name="c", subcore_axis_name="s"),
        out_type=jax.ShapeDtypeStruct((B * N, D), jnp.float32),
        scratch_types=[
            pltpu.VMEM((_NCHUNK, _CH), jnp.int32),
        ] + [pltpu.VMEM((_CH, D), jnp.float32)] * _NBUF
          + [pltpu.SemaphoreType.DMA] * (2 * _NBUF),
    )
    def _gather_rows(tab_hbm, idx_hbm, out_hbm,
                     idx_v, b0, b1, b2, g0, g1, g2, w0, w1, w2):
        _gather_kernel(tab_hbm, idx_hbm, out_hbm,
                       idx_v, (b0, b1, b2), (g0, g1, g2), (w0, w1, w2))

    return _gather_rows


def kernel(hit_embed, hit_phi, key_embed, key_phi):
    gather = _make_gather_rows()
    hit_sphi, hit_idx = _argsort_batch(hit_phi)
    hit_s = gather(hit_embed.reshape(B * N, D),
                   hit_idx.reshape((B * N) // _CH, _CH))
    key_sphi, key_idx = _argsort_batch(key_phi)
    key_s = gather(key_embed.reshape(B * N, D),
                   key_idx.reshape((B * N) // _CH, _CH))
    return (
        hit_s.reshape(B, N, D),
        hit_sphi,
        key_s.reshape(B, N, D),
        key_sphi,
    )
